# baseline (speedup 1.0000x reference)
import functools

import jax
import jax.numpy as jnp
from jax import lax
from jax.experimental import pallas as pl
from jax.experimental.pallas import tpu as pltpu
from jax.experimental.pallas import tpu_sc as plsc

N = 10000
E = 320000
D = 128
DH = D // 2

NC = 2
NS = 16
NW = NC * NS
CH = 80
EPW = E // NW
CPW = EPW // CH

ECH = 128
EPAD = 327680
NCHT = EPAD // ECH
CPT = NCHT // NS
NB = 4
EGRP = CPT // NB

NPAD = 10240
RPT = NPAD // NS

BLK = 400
GRID = N // BLK

_MESH = plsc.VectorSubcoreMesh(core_axis_name="c", subcore_axis_name="s")
_DEFPREC = jax.lax.Precision.DEFAULT
_SC_PARAMS = pltpu.CompilerParams(needs_layout_passes=False)
_SC_PARAMS_UNTILED = pltpu.CompilerParams(needs_layout_passes=False,
                                          use_tc_tiling_on_sc=False)


@functools.partial(
    pl.kernel,
    out_type=jax.ShapeDtypeStruct((2, NW, 1, N), jnp.float32),
    mesh=_MESH,
    scratch_types=[
        pltpu.VMEM((CPW, CH), jnp.int32),
        pltpu.VMEM((N,), jnp.float32),
    ],
    compiler_params=_SC_PARAMS,
)
def _deg_kernel(src_hbm, dst_hbm, out_hbm, idx_v, hist_v):
    c = lax.axis_index("c")
    s = lax.axis_index("s")
    wid = s * NC + c
    ones = jnp.full((16,), 1.0, dtype=jnp.float32)
    zeros = jnp.zeros((16,), dtype=jnp.float32)

    def one_direction(edge_hbm, out_row):
        pltpu.sync_copy(edge_hbm.at[wid], idx_v)

        def zbody(i, _):
            hist_v[pl.ds(i * 16, 16)] = zeros
            return _

        lax.fori_loop(0, N // 16, zbody, None)

        def hbody(r, _):
            for k in range(CH // 16):
                v = idx_v[r, pl.ds(k * 16, 16)]
                plsc.addupdate_scatter(hist_v, [v], ones)
            return _

        lax.fori_loop(0, CPW, hbody, None)
        pltpu.sync_copy(hist_v, out_row)

    one_direction(src_hbm, out_hbm.at[0, wid, 0])
    one_direction(dst_hbm, out_hbm.at[1, wid, 0])


@functools.partial(
    pl.kernel,
    out_type=jax.ShapeDtypeStruct((NC, NPAD, DH), jnp.float32),
    mesh=_MESH,
    scratch_types=[
        pltpu.VMEM((CPT, ECH), jnp.int32),
        pltpu.VMEM((CPT, ECH), jnp.int32),
        [pltpu.VMEM((ECH, DH), jnp.float32) for _ in range(NB)],
        pltpu.VMEM_SHARED((NPAD, DH), jnp.float32),
        [pltpu.SemaphoreType.DMA for _ in range(NB)],
        [pltpu.SemaphoreType.DMA for _ in range(NB)],
        pltpu.SemaphoreType.DMA,
    ],
    compiler_params=_SC_PARAMS_UNTILED,
)
def _edge_kernel(hs_hbm, src_hbm, dst_hbm, out_hbm, isrc, idst, bufs,
                 agg, gsems, ssems, zsem):
    c = lax.axis_index("c")
    s = lax.axis_index("s")
    hsc = hs_hbm.at[c]
    zeros = jnp.zeros((16,), dtype=jnp.float32)

    def zbody(r, _):
        for k in range(DH // 16):
            bufs[0][r, pl.ds(k * 16, 16)] = zeros
        return _

    lax.fori_loop(0, ECH, zbody, None)
    for j in range(RPT // ECH):
        pltpu.async_copy(bufs[0], agg.at[pl.ds(s * RPT + j * ECH, ECH)], zsem)
    pltpu.sync_copy(src_hbm.at[pl.ds(s * CPT, CPT)], isrc)
    pltpu.sync_copy(dst_hbm.at[pl.ds(s * CPT, CPT)], idst)
    for j in range(RPT // ECH):
        pltpu.make_async_copy(bufs[0], agg.at[pl.ds(s * RPT, ECH)], zsem).wait()
    plsc.subcore_barrier()

    def _wait_gather(c_, k):
        pltpu.make_async_copy(hsc.at[isrc.at[c_]], bufs[k], gsems[k]).wait()

    def _start_scatter(c_, k):
        pltpu.async_copy(bufs[k], agg.at[idst.at[c_]], ssems[k], add=True)

    def _wait_scatter_refill(c_, k):
        pltpu.make_async_copy(bufs[k], agg.at[idst.at[c_]], ssems[k]).wait()
        nxt = c_ + NB

        @pl.when(nxt < CPT)
        def _():
            pltpu.async_copy(hsc.at[isrc.at[nxt]], bufs[k], gsems[k])

    for k in range(NB):
        pltpu.async_copy(hsc.at[isrc.at[k]], bufs[k], gsems[k])

    def body(i, _):
        base = i * NB
        for k in range(NB):
            _wait_gather(base + k, k)
            _start_scatter(base + k, k)
            if k >= 2:
                _wait_scatter_refill(base + k - 2, k - 2)
        for k in (NB - 2, NB - 1):
            _wait_scatter_refill(base + k, k)
        return _

    lax.fori_loop(0, EGRP, body, None)
    plsc.subcore_barrier()

    pltpu.sync_copy(agg.at[pl.ds(s * RPT, RPT)], out_hbm.at[c, pl.ds(s * RPT, RPT)])


def _norms_body(deg_ref, out_ref):
    d = jnp.maximum(jnp.sum(deg_ref[...], axis=1), 1.0)
    r = jax.lax.rsqrt(d)
    r = r * (1.5 - 0.5 * d * r * r)
    out_ref[...] = r


def _norms_tc(deg):
    return pl.pallas_call(
        _norms_body,
        out_shape=jax.ShapeDtypeStruct((2, N), jnp.float32),
    )(deg)


def _mm_scale_body(x_ref, w_ref, ns_ref, out_ref):
    h = jnp.dot(x_ref[...], w_ref[...], precision=_DEFPREC,
                preferred_element_type=jnp.float32) * ns_ref[...]
    out_ref[0] = h[:, :DH]
    out_ref[1] = h[:, DH:]


def _mm_scale_tc(x, w, ns_col):
    return pl.pallas_call(
        _mm_scale_body,
        grid=(GRID,),
        in_specs=[
            pl.BlockSpec((BLK, D), lambda i: (i, 0)),
            pl.BlockSpec((D, D), lambda i: (0, 0)),
            pl.BlockSpec((BLK, 1), lambda i: (i, 0)),
        ],
        out_specs=pl.BlockSpec((NC, BLK, DH), lambda i: (0, i, 0)),
        out_shape=jax.ShapeDtypeStruct((NC, N, DH), jnp.float32),
    )(x, w, ns_col)


def _mid_body(p_ref, nd_ref, b_ref, w_ref, ns_ref, out_ref):
    agg = jnp.concatenate((p_ref[0], p_ref[1]), axis=1)
    h = jax.nn.relu(agg * nd_ref[...] + b_ref[...])
    h2 = jnp.dot(h, w_ref[...], precision=_DEFPREC,
                 preferred_element_type=jnp.float32) * ns_ref[...]
    out_ref[0] = h2[:, :DH]
    out_ref[1] = h2[:, DH:]


def _mid_tc(p, nd_col, b_row, w, ns_col):
    return pl.pallas_call(
        _mid_body,
        grid=(GRID,),
        in_specs=[
            pl.BlockSpec((NC, BLK, DH), lambda i: (0, i, 0)),
            pl.BlockSpec((BLK, 1), lambda i: (i, 0)),
            pl.BlockSpec((1, D), lambda i: (0, 0)),
            pl.BlockSpec((D, D), lambda i: (0, 0)),
            pl.BlockSpec((BLK, 1), lambda i: (i, 0)),
        ],
        out_specs=pl.BlockSpec((NC, BLK, DH), lambda i: (0, i, 0)),
        out_shape=jax.ShapeDtypeStruct((NC, N, DH), jnp.float32),
    )(p, nd_col, b_row, w, ns_col)


def _final_body(p_ref, nd_ref, b_ref, wc1_ref, bc1_ref, wc2_ref, bc2_ref,
                ne_ref, ge_ref, lg_ref, acc_ref):
    i = pl.program_id(0)
    agg = jnp.concatenate((p_ref[0], p_ref[1]), axis=1)
    ne = jax.nn.relu(agg * nd_ref[...] + b_ref[...])
    ne_ref[...] = ne

    @pl.when(i == 0)
    def _():
        acc_ref[...] = jnp.zeros_like(acc_ref)

    acc_ref[...] += jnp.sum(ne, axis=0, keepdims=True)

    @pl.when(i == GRID - 1)
    def _():
        ge = acc_ref[...] * (1.0 / N)
        ge_ref[...] = ge
        hc = jax.nn.relu(jnp.dot(ge, wc1_ref[...], precision=_DEFPREC,
                                 preferred_element_type=jnp.float32)
                         + bc1_ref[...])
        lg_ref[...] = (jnp.sum(hc * wc2_ref[...], axis=1, keepdims=True)
                       + bc2_ref[...])


def _final_tc(p, nd_col, b_row, wc1, bc1_row, wc2, bc2_row):
    return pl.pallas_call(
        _final_body,
        grid=(GRID,),
        in_specs=[
            pl.BlockSpec((NC, BLK, DH), lambda i: (0, i, 0)),
            pl.BlockSpec((BLK, 1), lambda i: (i, 0)),
            pl.BlockSpec((1, D), lambda i: (0, 0)),
            pl.BlockSpec((D, D), lambda i: (0, 0)),
            pl.BlockSpec((1, D), lambda i: (0, 0)),
            pl.BlockSpec((1, D), lambda i: (0, 0)),
            pl.BlockSpec((1, 1), lambda i: (0, 0)),
        ],
        out_specs=[
            pl.BlockSpec((BLK, D), lambda i: (i, 0)),
            pl.BlockSpec((1, D), lambda i: (0, 0)),
            pl.BlockSpec((1, 1), lambda i: (0, 0)),
        ],
        out_shape=[
            jax.ShapeDtypeStruct((N, D), jnp.float32),
            jax.ShapeDtypeStruct((1, D), jnp.float32),
            jax.ShapeDtypeStruct((1, 1), jnp.float32),
        ],
        scratch_shapes=[pltpu.VMEM((1, D), jnp.float32)],
    )(p, nd_col, b_row, wc1, bc1_row, wc2, bc2_row)


def kernel(x, edge_index, W1, b1, W2, b2, Wc1, bc1, Wc2, bc2):
    src = edge_index[0]
    dst = edge_index[1]
    src2 = src.reshape(NW, CPW, CH)
    dst2 = dst.reshape(NW, CPW, CH)
    npad_e = EPAD - E
    src3 = jnp.concatenate(
        [src, jnp.zeros((npad_e,), jnp.int32)]).reshape(NCHT, ECH)
    dst3 = jnp.concatenate(
        [dst, N + (jnp.arange(npad_e, dtype=jnp.int32) % (NPAD - N))]
    ).reshape(NCHT, ECH)

    deg = _deg_kernel(src2, dst2).reshape(2, NW, N)
    norms = _norms_tc(deg)
    ns_col = norms[0].reshape(N, 1)
    nd_col = norms[1].reshape(N, 1)
    b1r = b1.reshape(1, D)
    b2r = b2.reshape(1, D)
    bc1r = bc1.reshape(1, D)
    bc2r = bc2.reshape(1, 1)
    wc2r = Wc2.reshape(1, D)

    hs1 = _mm_scale_tc(x, W1, ns_col)
    p1 = _edge_kernel(hs1, src3, dst3)
    hs2 = _mid_tc(p1, nd_col, b1r, W2, ns_col)
    p2 = _edge_kernel(hs2, src3, dst3)
    node_emb, graph_emb, logits = _final_tc(p2, nd_col, b2r, Wc1, bc1r,
                                            wc2r, bc2r)
    return (node_emb, graph_emb, logits)


# speedup vs baseline: 1.0623x; 1.0623x over previous
import functools

import jax
import jax.numpy as jnp
from jax import lax
from jax.experimental import pallas as pl
from jax.experimental.pallas import tpu as pltpu
from jax.experimental.pallas import tpu_sc as plsc

N = 10000
E = 320000
D = 128
DH = D // 2

NC = 2
NS = 16
NW = NC * NS
CH = 80
EPW = E // NW
CPW = EPW // CH

ECH = 112
ECPW = 90
EPAD = NW * ECPW * ECH
NCHT = EPAD // ECH
NB = 2
EGRP = ECPW // NB

NPAD = 10240
RPT = NPAD // NS

BLK = 400
GRID = N // BLK

_MESH = plsc.VectorSubcoreMesh(core_axis_name="c", subcore_axis_name="s")
_DEFPREC = jax.lax.Precision.DEFAULT
_SC_PARAMS = pltpu.CompilerParams(needs_layout_passes=False)
_SC_PARAMS_UNTILED = pltpu.CompilerParams(needs_layout_passes=False,
                                          use_tc_tiling_on_sc=False)


@functools.partial(
    pl.kernel,
    out_type=jax.ShapeDtypeStruct((2, NW, 1, N), jnp.float32),
    mesh=_MESH,
    scratch_types=[
        pltpu.VMEM((CPW, CH), jnp.int32),
        pltpu.VMEM((N,), jnp.float32),
    ],
    compiler_params=_SC_PARAMS,
)
def _deg_kernel(src_hbm, dst_hbm, out_hbm, idx_v, hist_v):
    c = lax.axis_index("c")
    s = lax.axis_index("s")
    wid = s * NC + c
    ones = jnp.full((16,), 1.0, dtype=jnp.float32)
    zeros = jnp.zeros((16,), dtype=jnp.float32)

    def one_direction(edge_hbm, out_row):
        pltpu.sync_copy(edge_hbm.at[wid], idx_v)

        def zbody(i, _):
            hist_v[pl.ds(i * 16, 16)] = zeros
            return _

        lax.fori_loop(0, N // 16, zbody, None)

        def hbody(r, _):
            for k in range(CH // 16):
                v = idx_v[r, pl.ds(k * 16, 16)]
                plsc.addupdate_scatter(hist_v, [v], ones)
            return _

        lax.fori_loop(0, CPW, hbody, None)
        pltpu.sync_copy(hist_v, out_row)

    one_direction(src_hbm, out_hbm.at[0, wid, 0])
    one_direction(dst_hbm, out_hbm.at[1, wid, 0])


@functools.partial(
    pl.kernel,
    out_type=jax.ShapeDtypeStruct((NC, NPAD, D), jnp.float32),
    mesh=_MESH,
    scratch_types=[
        pltpu.VMEM((ECPW, ECH), jnp.int32),
        pltpu.VMEM((ECPW, ECH), jnp.int32),
        [pltpu.VMEM((ECH, D), jnp.float32) for _ in range(NB)],
        pltpu.VMEM_SHARED((NPAD, D), jnp.float32),
        [pltpu.SemaphoreType.DMA for _ in range(NB)],
        [pltpu.SemaphoreType.DMA for _ in range(NB)],
        pltpu.SemaphoreType.DMA,
    ],
    compiler_params=_SC_PARAMS_UNTILED,
)
def _edge_kernel(hs_hbm, src_hbm, dst_hbm, out_hbm, isrc, idst, bufs,
                 agg, gsems, ssems, zsem):
    c = lax.axis_index("c")
    s = lax.axis_index("s")
    wid = s * NC + c
    zeros = jnp.zeros((16,), dtype=jnp.float32)

    def zbody(r, _):
        for k in range(D // 16):
            bufs[0][r, pl.ds(k * 16, 16)] = zeros
        return _

    lax.fori_loop(0, ECH, zbody, None)
    off = 0
    for zr in [ECH] * (RPT // ECH) + [RPT % ECH]:
        pltpu.async_copy(bufs[0].at[pl.ds(0, zr)],
                         agg.at[pl.ds(s * RPT + off, zr)], zsem)
        off += zr
    pltpu.sync_copy(src_hbm.at[wid], isrc)
    pltpu.sync_copy(dst_hbm.at[wid], idst)
    for zr in [ECH] * (RPT // ECH) + [RPT % ECH]:
        pltpu.make_async_copy(bufs[0].at[pl.ds(0, zr)],
                              agg.at[pl.ds(s * RPT, zr)], zsem).wait()
    plsc.subcore_barrier()

    def _wait_gather(c_, k):
        pltpu.make_async_copy(hs_hbm.at[isrc.at[c_]], bufs[k], gsems[k]).wait()

    def _start_scatter(c_, k):
        pltpu.async_copy(bufs[k], agg.at[idst.at[c_]], ssems[k], add=True)

    def _wait_scatter_refill(c_, k):
        pltpu.make_async_copy(bufs[k], agg.at[idst.at[c_]], ssems[k]).wait()
        nxt = c_ + NB

        @pl.when(nxt < ECPW)
        def _():
            pltpu.async_copy(hs_hbm.at[isrc.at[nxt]], bufs[k], gsems[k])

    for k in range(NB):
        pltpu.async_copy(hs_hbm.at[isrc.at[k]], bufs[k], gsems[k])

    def body(i, _):
        base = i * NB
        for k in range(NB):
            _wait_gather(base + k, k)
            _start_scatter(base + k, k)
        for k in range(NB):
            _wait_scatter_refill(base + k, k)
        return _

    lax.fori_loop(0, EGRP, body, None)
    plsc.subcore_barrier()

    pltpu.sync_copy(agg.at[pl.ds(s * RPT, RPT)], out_hbm.at[c, pl.ds(s * RPT, RPT)])


def _norms_body(deg_ref, out_ref):
    d = jnp.maximum(jnp.sum(deg_ref[...], axis=1), 1.0)
    r = jax.lax.rsqrt(d)
    r = r * (1.5 - 0.5 * d * r * r)
    out_ref[...] = r


def _norms_tc(deg):
    return pl.pallas_call(
        _norms_body,
        out_shape=jax.ShapeDtypeStruct((2, N), jnp.float32),
    )(deg)


def _mm_scale_body(x_ref, w_ref, ns_ref, out_ref):
    out_ref[...] = jnp.dot(x_ref[...], w_ref[...], precision=_DEFPREC,
                           preferred_element_type=jnp.float32) * ns_ref[...]


def _mm_scale_tc(x, w, ns_col):
    return pl.pallas_call(
        _mm_scale_body,
        grid=(GRID,),
        in_specs=[
            pl.BlockSpec((BLK, D), lambda i: (i, 0)),
            pl.BlockSpec((D, D), lambda i: (0, 0)),
            pl.BlockSpec((BLK, 1), lambda i: (i, 0)),
        ],
        out_specs=pl.BlockSpec((BLK, D), lambda i: (i, 0)),
        out_shape=jax.ShapeDtypeStruct((N, D), jnp.float32),
    )(x, w, ns_col)


def _mid_body(p_ref, nd_ref, b_ref, w_ref, ns_ref, out_ref):
    agg = p_ref[0] + p_ref[1]
    h = jax.nn.relu(agg * nd_ref[...] + b_ref[...])
    out_ref[...] = jnp.dot(h, w_ref[...], precision=_DEFPREC,
                           preferred_element_type=jnp.float32) * ns_ref[...]


def _mid_tc(p, nd_col, b_row, w, ns_col):
    return pl.pallas_call(
        _mid_body,
        grid=(GRID,),
        in_specs=[
            pl.BlockSpec((NC, BLK, D), lambda i: (0, i, 0)),
            pl.BlockSpec((BLK, 1), lambda i: (i, 0)),
            pl.BlockSpec((1, D), lambda i: (0, 0)),
            pl.BlockSpec((D, D), lambda i: (0, 0)),
            pl.BlockSpec((BLK, 1), lambda i: (i, 0)),
        ],
        out_specs=pl.BlockSpec((BLK, D), lambda i: (i, 0)),
        out_shape=jax.ShapeDtypeStruct((N, D), jnp.float32),
    )(p, nd_col, b_row, w, ns_col)


def _final_body(p_ref, nd_ref, b_ref, wc1_ref, bc1_ref, wc2_ref, bc2_ref,
                ne_ref, ge_ref, lg_ref, acc_ref):
    i = pl.program_id(0)
    agg = p_ref[0] + p_ref[1]
    ne = jax.nn.relu(agg * nd_ref[...] + b_ref[...])
    ne_ref[...] = ne

    @pl.when(i == 0)
    def _():
        acc_ref[...] = jnp.zeros_like(acc_ref)

    acc_ref[...] += jnp.sum(ne, axis=0, keepdims=True)

    @pl.when(i == GRID - 1)
    def _():
        ge = acc_ref[...] * (1.0 / N)
        ge_ref[...] = ge
        hc = jax.nn.relu(jnp.dot(ge, wc1_ref[...], precision=_DEFPREC,
                                 preferred_element_type=jnp.float32)
                         + bc1_ref[...])
        lg_ref[...] = (jnp.sum(hc * wc2_ref[...], axis=1, keepdims=True)
                       + bc2_ref[...])


def _final_tc(p, nd_col, b_row, wc1, bc1_row, wc2, bc2_row):
    return pl.pallas_call(
        _final_body,
        grid=(GRID,),
        in_specs=[
            pl.BlockSpec((NC, BLK, D), lambda i: (0, i, 0)),
            pl.BlockSpec((BLK, 1), lambda i: (i, 0)),
            pl.BlockSpec((1, D), lambda i: (0, 0)),
            pl.BlockSpec((D, D), lambda i: (0, 0)),
            pl.BlockSpec((1, D), lambda i: (0, 0)),
            pl.BlockSpec((1, D), lambda i: (0, 0)),
            pl.BlockSpec((1, 1), lambda i: (0, 0)),
        ],
        out_specs=[
            pl.BlockSpec((BLK, D), lambda i: (i, 0)),
            pl.BlockSpec((1, D), lambda i: (0, 0)),
            pl.BlockSpec((1, 1), lambda i: (0, 0)),
        ],
        out_shape=[
            jax.ShapeDtypeStruct((N, D), jnp.float32),
            jax.ShapeDtypeStruct((1, D), jnp.float32),
            jax.ShapeDtypeStruct((1, 1), jnp.float32),
        ],
        scratch_shapes=[pltpu.VMEM((1, D), jnp.float32)],
    )(p, nd_col, b_row, wc1, bc1_row, wc2, bc2_row)


def kernel(x, edge_index, W1, b1, W2, b2, Wc1, bc1, Wc2, bc2):
    src = edge_index[0]
    dst = edge_index[1]
    src2 = src.reshape(NW, CPW, CH)
    dst2 = dst.reshape(NW, CPW, CH)
    npad_e = EPAD - E
    src3 = jnp.concatenate(
        [src, jnp.zeros((npad_e,), jnp.int32)]).reshape(NW, ECPW, ECH)
    dst3 = jnp.concatenate(
        [dst, N + (jnp.arange(npad_e, dtype=jnp.int32) % (NPAD - N))]
    ).reshape(NW, ECPW, ECH)

    deg = _deg_kernel(src2, dst2).reshape(2, NW, N)
    norms = _norms_tc(deg)
    ns_col = norms[0].reshape(N, 1)
    nd_col = norms[1].reshape(N, 1)
    b1r = b1.reshape(1, D)
    b2r = b2.reshape(1, D)
    bc1r = bc1.reshape(1, D)
    bc2r = bc2.reshape(1, 1)
    wc2r = Wc2.reshape(1, D)

    hs1 = _mm_scale_tc(x, W1, ns_col)
    p1 = _edge_kernel(hs1, src3, dst3)
    hs2 = _mid_tc(p1, nd_col, b1r, W2, ns_col)
    p2 = _edge_kernel(hs2, src3, dst3)
    node_emb, graph_emb, logits = _final_tc(p2, nd_col, b2r, Wc1, bc1r,
                                            wc2r, bc2r)
    return (node_emb, graph_emb, logits)


# speedup vs baseline: 1.7170x; 1.6163x over previous
import functools

import jax
import jax.numpy as jnp
from jax import lax
from jax.experimental import pallas as pl
from jax.experimental.pallas import tpu as pltpu
from jax.experimental.pallas import tpu_sc as plsc

N = 10000
E = 320000
D = 128
DH = D // 2

NC = 2
NS = 16
NW = NC * NS
CH = 80
EPW = E // NW
CPW = EPW // CH

ECH = 112
ECPW = 90
EPAD = NW * ECPW * ECH
NCHT = EPAD // ECH
NB = 2
EGRP = ECPW // NB

NPAD = 10240
RPT = NPAD // NS

BLK = 400
GRID = N // BLK

_MESH = plsc.VectorSubcoreMesh(core_axis_name="c", subcore_axis_name="s")
_DEFPREC = jax.lax.Precision.DEFAULT
_SC_PARAMS = pltpu.CompilerParams(needs_layout_passes=False)
_SC_PARAMS_UNTILED = pltpu.CompilerParams(needs_layout_passes=False,
                                          use_tc_tiling_on_sc=False)


@functools.partial(
    pl.kernel,
    out_type=jax.ShapeDtypeStruct((2, NW, 1, N), jnp.float32),
    mesh=_MESH,
    scratch_types=[
        pltpu.VMEM((CPW, CH), jnp.int32),
        pltpu.VMEM((N,), jnp.float32),
    ],
    compiler_params=_SC_PARAMS,
)
def _deg_kernel(src_hbm, dst_hbm, out_hbm, idx_v, hist_v):
    c = lax.axis_index("c")
    s = lax.axis_index("s")
    wid = s * NC + c
    ones = jnp.full((16,), 1.0, dtype=jnp.float32)
    zeros = jnp.zeros((16,), dtype=jnp.float32)

    def one_direction(edge_hbm, out_row):
        pltpu.sync_copy(edge_hbm.at[wid], idx_v)

        def zbody(i, _):
            hist_v[pl.ds(i * 16, 16)] = zeros
            return _

        lax.fori_loop(0, N // 16, zbody, None)

        def hbody(r, _):
            for k in range(CH // 16):
                v = idx_v[r, pl.ds(k * 16, 16)]
                plsc.addupdate_scatter(hist_v, [v], ones)
            return _

        lax.fori_loop(0, CPW, hbody, None)
        pltpu.sync_copy(hist_v, out_row)

    one_direction(src_hbm, out_hbm.at[0, wid, 0])
    one_direction(dst_hbm, out_hbm.at[1, wid, 0])


@functools.partial(
    pl.kernel,
    out_type=jax.ShapeDtypeStruct((NC, NPAD, D), jnp.float32),
    mesh=_MESH,
    scratch_types=[
        pltpu.VMEM((ECPW, ECH), jnp.int32),
        pltpu.VMEM((ECPW, ECH), jnp.int32),
        [pltpu.VMEM((ECH, D), jnp.float32) for _ in range(NB)],
        pltpu.VMEM_SHARED((NPAD, D), jnp.float32),
        [pltpu.SemaphoreType.DMA for _ in range(NB)],
        [pltpu.SemaphoreType.DMA for _ in range(NB)],
        pltpu.SemaphoreType.DMA,
    ],
    compiler_params=_SC_PARAMS_UNTILED,
)
def _edge_kernel(hs_hbm, src_hbm, dst_hbm, out_hbm, isrc, idst, bufs,
                 agg, gsems, ssems, zsem):
    c = lax.axis_index("c")
    s = lax.axis_index("s")
    wid = s * NC + c
    zeros = jnp.zeros((16,), dtype=jnp.float32)

    def zbody(r, _):
        for k in range(D // 16):
            bufs[0][r, pl.ds(k * 16, 16)] = zeros
        return _

    lax.fori_loop(0, ECH, zbody, None)
    off = 0
    for zr in [ECH] * (RPT // ECH) + [RPT % ECH]:
        pltpu.async_copy(bufs[0].at[pl.ds(0, zr)],
                         agg.at[pl.ds(s * RPT + off, zr)], zsem)
        off += zr
    pltpu.sync_copy(src_hbm.at[wid], isrc)
    pltpu.sync_copy(dst_hbm.at[wid], idst)
    for zr in [ECH] * (RPT // ECH) + [RPT % ECH]:
        pltpu.make_async_copy(bufs[0].at[pl.ds(0, zr)],
                              agg.at[pl.ds(s * RPT, zr)], zsem).wait()
    plsc.subcore_barrier()

    def _wait_gather(c_, k):
        pltpu.make_async_copy(hs_hbm.at[isrc.at[c_]], bufs[k], gsems[k]).wait()

    def _start_scatter(c_, k):
        pltpu.async_copy(bufs[k], agg.at[idst.at[c_]], ssems[k], add=True)

    def _wait_scatter_refill(c_, k):
        pltpu.make_async_copy(bufs[k], agg.at[idst.at[c_]], ssems[k]).wait()
        nxt = c_ + NB

        @pl.when(nxt < ECPW)
        def _():
            pltpu.async_copy(hs_hbm.at[isrc.at[nxt]], bufs[k], gsems[k])

    for k in range(NB):
        pltpu.async_copy(hs_hbm.at[isrc.at[k]], bufs[k], gsems[k])

    def body(i, _):
        base = i * NB
        for k in range(NB):
            _wait_gather(base + k, k)
            _start_scatter(base + k, k)
        for k in range(NB):
            _wait_scatter_refill(base + k, k)
        return _

    lax.fori_loop(0, EGRP, body, None)
    plsc.subcore_barrier()

    pltpu.sync_copy(agg.at[pl.ds(s * RPT, RPT)], out_hbm.at[c, pl.ds(s * RPT, RPT)])


def _norms_body(deg_ref, out_ref):
    d = jnp.maximum(jnp.sum(deg_ref[...], axis=1), 1.0)
    r = jax.lax.rsqrt(d)
    r = r * (1.5 - 0.5 * d * r * r)
    out_ref[...] = r


def _norms_tc(deg):
    return pl.pallas_call(
        _norms_body,
        out_shape=jax.ShapeDtypeStruct((2, N), jnp.float32),
    )(deg)


def _mm_scale_body(x_ref, w_ref, ns_ref, out_ref):
    out_ref[...] = jnp.dot(x_ref[...], w_ref[...], precision=_DEFPREC,
                           preferred_element_type=jnp.float32) * ns_ref[...]


def _mm_scale_tc(x, w, ns_col):
    return pl.pallas_call(
        _mm_scale_body,
        grid=(GRID,),
        in_specs=[
            pl.BlockSpec((BLK, D), lambda i: (i, 0)),
            pl.BlockSpec((D, D), lambda i: (0, 0)),
            pl.BlockSpec((BLK, 1), lambda i: (i, 0)),
        ],
        out_specs=pl.BlockSpec((BLK, D), lambda i: (i, 0)),
        out_shape=jax.ShapeDtypeStruct((N, D), jnp.float32),
    )(x, w, ns_col)


def _mid_body(p_ref, nd_ref, b_ref, w_ref, ns_ref, out_ref):
    agg = p_ref[0] + p_ref[1]
    h = jax.nn.relu(agg * nd_ref[...] + b_ref[...])
    out_ref[...] = jnp.dot(h, w_ref[...], precision=_DEFPREC,
                           preferred_element_type=jnp.float32) * ns_ref[...]


def _mid_tc(p, nd_col, b_row, w, ns_col):
    return pl.pallas_call(
        _mid_body,
        grid=(GRID,),
        in_specs=[
            pl.BlockSpec((NC, BLK, D), lambda i: (0, i, 0)),
            pl.BlockSpec((BLK, 1), lambda i: (i, 0)),
            pl.BlockSpec((1, D), lambda i: (0, 0)),
            pl.BlockSpec((D, D), lambda i: (0, 0)),
            pl.BlockSpec((BLK, 1), lambda i: (i, 0)),
        ],
        out_specs=pl.BlockSpec((BLK, D), lambda i: (i, 0)),
        out_shape=jax.ShapeDtypeStruct((N, D), jnp.float32),
    )(p, nd_col, b_row, w, ns_col)


def _final_body(p_ref, nd_ref, b_ref, wc1_ref, bc1_ref, wc2_ref, bc2_ref,
                ne_ref, ge_ref, lg_ref, acc_ref):
    i = pl.program_id(0)
    agg = p_ref[0] + p_ref[1]
    ne = jax.nn.relu(agg * nd_ref[...] + b_ref[...])
    ne_ref[...] = ne

    @pl.when(i == 0)
    def _():
        acc_ref[...] = jnp.zeros_like(acc_ref)

    acc_ref[...] += jnp.sum(ne, axis=0, keepdims=True)

    @pl.when(i == GRID - 1)
    def _():
        ge = acc_ref[...] * (1.0 / N)
        ge_ref[...] = ge
        hc = jax.nn.relu(jnp.dot(ge, wc1_ref[...], precision=_DEFPREC,
                                 preferred_element_type=jnp.float32)
                         + bc1_ref[...])
        lg_ref[...] = (jnp.sum(hc * wc2_ref[...], axis=1, keepdims=True)
                       + bc2_ref[...])


def _final_tc(p, nd_col, b_row, wc1, bc1_row, wc2, bc2_row):
    return pl.pallas_call(
        _final_body,
        grid=(GRID,),
        in_specs=[
            pl.BlockSpec((NC, BLK, D), lambda i: (0, i, 0)),
            pl.BlockSpec((BLK, 1), lambda i: (i, 0)),
            pl.BlockSpec((1, D), lambda i: (0, 0)),
            pl.BlockSpec((D, D), lambda i: (0, 0)),
            pl.BlockSpec((1, D), lambda i: (0, 0)),
            pl.BlockSpec((1, D), lambda i: (0, 0)),
            pl.BlockSpec((1, 1), lambda i: (0, 0)),
        ],
        out_specs=[
            pl.BlockSpec((BLK, D), lambda i: (i, 0)),
            pl.BlockSpec((1, D), lambda i: (0, 0)),
            pl.BlockSpec((1, 1), lambda i: (0, 0)),
        ],
        out_shape=[
            jax.ShapeDtypeStruct((N, D), jnp.float32),
            jax.ShapeDtypeStruct((1, D), jnp.float32),
            jax.ShapeDtypeStruct((1, 1), jnp.float32),
        ],
        scratch_shapes=[pltpu.VMEM((1, D), jnp.float32)],
    )(p, nd_col, b_row, wc1, bc1_row, wc2, bc2_row)


def kernel(x, edge_index, W1, b1, W2, b2, Wc1, bc1, Wc2, bc2):
    src = edge_index[0]
    dst = edge_index[1]
    src2 = src.reshape(NW, CPW, CH)
    dst2 = dst.reshape(NW, CPW, CH)
    npad_e = EPAD - E
    src3 = jnp.concatenate(
        [src, (jnp.arange(npad_e, dtype=jnp.int32) * 37) % N]
    ).reshape(NW, ECPW, ECH)
    dst3 = jnp.concatenate(
        [dst, N + (jnp.arange(npad_e, dtype=jnp.int32) % (NPAD - N))]
    ).reshape(NW, ECPW, ECH)

    deg = _deg_kernel(src2, dst2).reshape(2, NW, N)
    norms = _norms_tc(deg)
    ns_col = norms[0].reshape(N, 1)
    nd_col = norms[1].reshape(N, 1)
    b1r = b1.reshape(1, D)
    b2r = b2.reshape(1, D)
    bc1r = bc1.reshape(1, D)
    bc2r = bc2.reshape(1, 1)
    wc2r = Wc2.reshape(1, D)

    hs1 = _mm_scale_tc(x, W1, ns_col)
    p1 = _edge_kernel(hs1, src3, dst3)
    hs2 = _mid_tc(p1, nd_col, b1r, W2, ns_col)
    p2 = _edge_kernel(hs2, src3, dst3)
    node_emb, graph_emb, logits = _final_tc(p2, nd_col, b2r, Wc1, bc1r,
                                            wc2r, bc2r)
    return (node_emb, graph_emb, logits)


# speedup vs baseline: 1.9451x; 1.1329x over previous
import functools

import jax
import jax.numpy as jnp
from jax import lax
from jax.experimental import pallas as pl
from jax.experimental.pallas import tpu as pltpu
from jax.experimental.pallas import tpu_sc as plsc

N = 10000
E = 320000
D = 128
DH = D // 2

NC = 2
NS = 16
NW = NC * NS
CH = 80
EPW = E // NW
CPW = EPW // CH

ECH = 72
ECPW = 139
EPAD = NW * ECPW * ECH
NCHT = EPAD // ECH
NB = 3
EGRP = ECPW // NB

NPAD = 10240
RPT = NPAD // NS

BLK = 400
GRID = N // BLK

_MESH = plsc.VectorSubcoreMesh(core_axis_name="c", subcore_axis_name="s")
_DEFPREC = jax.lax.Precision.DEFAULT
_SC_PARAMS = pltpu.CompilerParams(needs_layout_passes=False)
_SC_PARAMS_UNTILED = pltpu.CompilerParams(needs_layout_passes=False,
                                          use_tc_tiling_on_sc=False)


@functools.partial(
    pl.kernel,
    out_type=jax.ShapeDtypeStruct((2, NW, 1, N), jnp.float32),
    mesh=_MESH,
    scratch_types=[
        pltpu.VMEM((CPW, CH), jnp.int32),
        pltpu.VMEM((N,), jnp.float32),
    ],
    compiler_params=_SC_PARAMS,
)
def _deg_kernel(src_hbm, dst_hbm, out_hbm, idx_v, hist_v):
    c = lax.axis_index("c")
    s = lax.axis_index("s")
    wid = s * NC + c
    ones = jnp.full((16,), 1.0, dtype=jnp.float32)
    zeros = jnp.zeros((16,), dtype=jnp.float32)

    def one_direction(edge_hbm, out_row):
        pltpu.sync_copy(edge_hbm.at[wid], idx_v)

        def zbody(i, _):
            hist_v[pl.ds(i * 16, 16)] = zeros
            return _

        lax.fori_loop(0, N // 16, zbody, None)

        def hbody(r, _):
            for k in range(CH // 16):
                v = idx_v[r, pl.ds(k * 16, 16)]
                plsc.addupdate_scatter(hist_v, [v], ones)
            return _

        lax.fori_loop(0, CPW, hbody, None)
        pltpu.sync_copy(hist_v, out_row)

    one_direction(src_hbm, out_hbm.at[0, wid, 0])
    one_direction(dst_hbm, out_hbm.at[1, wid, 0])


@functools.partial(
    pl.kernel,
    out_type=jax.ShapeDtypeStruct((NC, NPAD, D), jnp.float32),
    mesh=_MESH,
    scratch_types=[
        pltpu.VMEM((ECPW, ECH), jnp.int32),
        pltpu.VMEM((ECPW, ECH), jnp.int32),
        [pltpu.VMEM((ECH, D), jnp.float32) for _ in range(NB)],
        pltpu.VMEM_SHARED((NPAD, D), jnp.float32),
        [pltpu.SemaphoreType.DMA for _ in range(NB)],
        [pltpu.SemaphoreType.DMA for _ in range(NB)],
        pltpu.SemaphoreType.DMA,
    ],
    compiler_params=_SC_PARAMS_UNTILED,
)
def _edge_kernel(hs_hbm, src_hbm, dst_hbm, out_hbm, isrc, idst, bufs,
                 agg, gsems, ssems, zsem):
    c = lax.axis_index("c")
    s = lax.axis_index("s")
    wid = s * NC + c
    zeros = jnp.zeros((16,), dtype=jnp.float32)

    def zbody(r, _):
        for k in range(D // 16):
            bufs[0][r, pl.ds(k * 16, 16)] = zeros
        return _

    lax.fori_loop(0, ECH, zbody, None)
    off = 0
    for zr in [ECH] * (RPT // ECH) + [RPT % ECH]:
        pltpu.async_copy(bufs[0].at[pl.ds(0, zr)],
                         agg.at[pl.ds(s * RPT + off, zr)], zsem)
        off += zr
    pltpu.sync_copy(src_hbm.at[wid], isrc)
    pltpu.sync_copy(dst_hbm.at[wid], idst)
    for zr in [ECH] * (RPT // ECH) + [RPT % ECH]:
        pltpu.make_async_copy(bufs[0].at[pl.ds(0, zr)],
                              agg.at[pl.ds(s * RPT, zr)], zsem).wait()
    plsc.subcore_barrier()

    def _wait_gather(c_, k):
        pltpu.make_async_copy(hs_hbm.at[isrc.at[c_]], bufs[k], gsems[k]).wait()

    def _start_scatter(c_, k):
        pltpu.async_copy(bufs[k], agg.at[idst.at[c_]], ssems[k], add=True)

    def _wait_scatter_refill(c_, k):
        pltpu.make_async_copy(bufs[k], agg.at[idst.at[c_]], ssems[k]).wait()
        nxt = c_ + NB

        @pl.when(nxt < ECPW)
        def _():
            pltpu.async_copy(hs_hbm.at[isrc.at[nxt]], bufs[k], gsems[k])

    for k in range(NB):
        pltpu.async_copy(hs_hbm.at[isrc.at[k]], bufs[k], gsems[k])

    def body(i, _):
        base = i * NB
        for k in range(NB):
            _wait_gather(base + k, k)
            _start_scatter(base + k, k)
            if k >= NB - 1:
                _wait_scatter_refill(base + k - (NB - 1), k - (NB - 1))
        for k in range(1, NB):
            _wait_scatter_refill(base + k, k)
        return _

    lax.fori_loop(0, EGRP, body, None)

    for k in range(ECPW - EGRP * NB):
        c_ = EGRP * NB + k
        _wait_gather(c_, k)
        pltpu.sync_copy(bufs[k], agg.at[idst.at[c_]], add=True)
    plsc.subcore_barrier()

    pltpu.sync_copy(agg.at[pl.ds(s * RPT, RPT)], out_hbm.at[c, pl.ds(s * RPT, RPT)])


def _norms_body(deg_ref, out_ref):
    d = jnp.maximum(jnp.sum(deg_ref[...], axis=1), 1.0)
    r = jax.lax.rsqrt(d)
    r = r * (1.5 - 0.5 * d * r * r)
    out_ref[...] = r


def _norms_tc(deg):
    return pl.pallas_call(
        _norms_body,
        out_shape=jax.ShapeDtypeStruct((2, N), jnp.float32),
    )(deg)


def _mm_scale_body(x_ref, w_ref, ns_ref, out_ref):
    out_ref[...] = jnp.dot(x_ref[...], w_ref[...], precision=_DEFPREC,
                           preferred_element_type=jnp.float32) * ns_ref[...]


def _mm_scale_tc(x, w, ns_col):
    return pl.pallas_call(
        _mm_scale_body,
        grid=(GRID,),
        in_specs=[
            pl.BlockSpec((BLK, D), lambda i: (i, 0)),
            pl.BlockSpec((D, D), lambda i: (0, 0)),
            pl.BlockSpec((BLK, 1), lambda i: (i, 0)),
        ],
        out_specs=pl.BlockSpec((BLK, D), lambda i: (i, 0)),
        out_shape=jax.ShapeDtypeStruct((N, D), jnp.float32),
    )(x, w, ns_col)


def _mid_body(p_ref, nd_ref, b_ref, w_ref, ns_ref, out_ref):
    agg = p_ref[0] + p_ref[1]
    h = jax.nn.relu(agg * nd_ref[...] + b_ref[...])
    out_ref[...] = jnp.dot(h, w_ref[...], precision=_DEFPREC,
                           preferred_element_type=jnp.float32) * ns_ref[...]


def _mid_tc(p, nd_col, b_row, w, ns_col):
    return pl.pallas_call(
        _mid_body,
        grid=(GRID,),
        in_specs=[
            pl.BlockSpec((NC, BLK, D), lambda i: (0, i, 0)),
            pl.BlockSpec((BLK, 1), lambda i: (i, 0)),
            pl.BlockSpec((1, D), lambda i: (0, 0)),
            pl.BlockSpec((D, D), lambda i: (0, 0)),
            pl.BlockSpec((BLK, 1), lambda i: (i, 0)),
        ],
        out_specs=pl.BlockSpec((BLK, D), lambda i: (i, 0)),
        out_shape=jax.ShapeDtypeStruct((N, D), jnp.float32),
    )(p, nd_col, b_row, w, ns_col)


def _final_body(p_ref, nd_ref, b_ref, wc1_ref, bc1_ref, wc2_ref, bc2_ref,
                ne_ref, ge_ref, lg_ref, acc_ref):
    i = pl.program_id(0)
    agg = p_ref[0] + p_ref[1]
    ne = jax.nn.relu(agg * nd_ref[...] + b_ref[...])
    ne_ref[...] = ne

    @pl.when(i == 0)
    def _():
        acc_ref[...] = jnp.zeros_like(acc_ref)

    acc_ref[...] += jnp.sum(ne, axis=0, keepdims=True)

    @pl.when(i == GRID - 1)
    def _():
        ge = acc_ref[...] * (1.0 / N)
        ge_ref[...] = ge
        hc = jax.nn.relu(jnp.dot(ge, wc1_ref[...], precision=_DEFPREC,
                                 preferred_element_type=jnp.float32)
                         + bc1_ref[...])
        lg_ref[...] = (jnp.sum(hc * wc2_ref[...], axis=1, keepdims=True)
                       + bc2_ref[...])


def _final_tc(p, nd_col, b_row, wc1, bc1_row, wc2, bc2_row):
    return pl.pallas_call(
        _final_body,
        grid=(GRID,),
        in_specs=[
            pl.BlockSpec((NC, BLK, D), lambda i: (0, i, 0)),
            pl.BlockSpec((BLK, 1), lambda i: (i, 0)),
            pl.BlockSpec((1, D), lambda i: (0, 0)),
            pl.BlockSpec((D, D), lambda i: (0, 0)),
            pl.BlockSpec((1, D), lambda i: (0, 0)),
            pl.BlockSpec((1, D), lambda i: (0, 0)),
            pl.BlockSpec((1, 1), lambda i: (0, 0)),
        ],
        out_specs=[
            pl.BlockSpec((BLK, D), lambda i: (i, 0)),
            pl.BlockSpec((1, D), lambda i: (0, 0)),
            pl.BlockSpec((1, 1), lambda i: (0, 0)),
        ],
        out_shape=[
            jax.ShapeDtypeStruct((N, D), jnp.float32),
            jax.ShapeDtypeStruct((1, D), jnp.float32),
            jax.ShapeDtypeStruct((1, 1), jnp.float32),
        ],
        scratch_shapes=[pltpu.VMEM((1, D), jnp.float32)],
    )(p, nd_col, b_row, wc1, bc1_row, wc2, bc2_row)


def kernel(x, edge_index, W1, b1, W2, b2, Wc1, bc1, Wc2, bc2):
    src = edge_index[0]
    dst = edge_index[1]
    src2 = src.reshape(NW, CPW, CH)
    dst2 = dst.reshape(NW, CPW, CH)
    npad_e = EPAD - E
    src3 = jnp.concatenate(
        [src, (jnp.arange(npad_e, dtype=jnp.int32) * 37) % N]
    ).reshape(NW, ECPW, ECH)
    dst3 = jnp.concatenate(
        [dst, N + (jnp.arange(npad_e, dtype=jnp.int32) % (NPAD - N))]
    ).reshape(NW, ECPW, ECH)

    deg = _deg_kernel(src2, dst2).reshape(2, NW, N)
    norms = _norms_tc(deg)
    ns_col = norms[0].reshape(N, 1)
    nd_col = norms[1].reshape(N, 1)
    b1r = b1.reshape(1, D)
    b2r = b2.reshape(1, D)
    bc1r = bc1.reshape(1, D)
    bc2r = bc2.reshape(1, 1)
    wc2r = Wc2.reshape(1, D)

    hs1 = _mm_scale_tc(x, W1, ns_col)
    p1 = _edge_kernel(hs1, src3, dst3)
    hs2 = _mid_tc(p1, nd_col, b1r, W2, ns_col)
    p2 = _edge_kernel(hs2, src3, dst3)
    node_emb, graph_emb, logits = _final_tc(p2, nd_col, b2r, Wc1, bc1r,
                                            wc2r, bc2r)
    return (node_emb, graph_emb, logits)


# speedup vs baseline: 1.9668x; 1.0111x over previous
import functools

import jax
import jax.numpy as jnp
from jax import lax
from jax.experimental import pallas as pl
from jax.experimental.pallas import tpu as pltpu
from jax.experimental.pallas import tpu_sc as plsc

N = 10000
E = 320000
D = 128
DH = D // 2

NC = 2
NS = 16
NW = NC * NS
CH = 80
EPW = E // NW
CPW = EPW // CH

ECH = 72
ECPW = 139
EPAD = NW * ECPW * ECH
NCHT = EPAD // ECH
NB = 3
EGRP = ECPW // NB

NPAD = 10240
RPT = NPAD // NS

BLK = 400
GRID = N // BLK

_MESH = plsc.VectorSubcoreMesh(core_axis_name="c", subcore_axis_name="s")
_DEFPREC = jax.lax.Precision.DEFAULT
_SC_PARAMS = pltpu.CompilerParams(needs_layout_passes=False)
_SC_PARAMS_UNTILED = pltpu.CompilerParams(needs_layout_passes=False,
                                          use_tc_tiling_on_sc=False)


@functools.partial(
    pl.kernel,
    out_type=jax.ShapeDtypeStruct((2, NW, N), jnp.float32),
    mesh=_MESH,
    scratch_types=[
        pltpu.VMEM((CPW, CH), jnp.int32),
        pltpu.VMEM((N,), jnp.float32),
    ],
    compiler_params=_SC_PARAMS_UNTILED,
)
def _deg_kernel(src_hbm, dst_hbm, out_hbm, idx_v, hist_v):
    c = lax.axis_index("c")
    s = lax.axis_index("s")
    wid = s * NC + c
    ones = jnp.full((16,), 1.0, dtype=jnp.float32)
    zeros = jnp.zeros((16,), dtype=jnp.float32)

    def one_direction(edge_hbm, out_row):
        pltpu.sync_copy(edge_hbm.at[wid], idx_v)

        def zbody(i, _):
            hist_v[pl.ds(i * 16, 16)] = zeros
            return _

        lax.fori_loop(0, N // 16, zbody, None)

        def hbody(r, _):
            for k in range(CH // 16):
                v = idx_v[r, pl.ds(k * 16, 16)]
                plsc.addupdate_scatter(hist_v, [v], ones)
            return _

        lax.fori_loop(0, CPW, hbody, None)
        pltpu.sync_copy(hist_v, out_row)

    one_direction(src_hbm, out_hbm.at[0, wid])
    one_direction(dst_hbm, out_hbm.at[1, wid])


@functools.partial(
    pl.kernel,
    out_type=jax.ShapeDtypeStruct((NC, NPAD, D), jnp.float32),
    mesh=_MESH,
    scratch_types=[
        pltpu.VMEM((ECPW, ECH), jnp.int32),
        pltpu.VMEM((ECPW, ECH), jnp.int32),
        [pltpu.VMEM((ECH, D), jnp.float32) for _ in range(NB)],
        pltpu.VMEM_SHARED((NPAD, D), jnp.float32),
        [pltpu.SemaphoreType.DMA for _ in range(NB)],
        [pltpu.SemaphoreType.DMA for _ in range(NB)],
        pltpu.SemaphoreType.DMA,
    ],
    compiler_params=_SC_PARAMS_UNTILED,
)
def _edge_kernel(hs_hbm, src_hbm, dst_hbm, out_hbm, isrc, idst, bufs,
                 agg, gsems, ssems, zsem):
    c = lax.axis_index("c")
    s = lax.axis_index("s")
    wid = s * NC + c
    zeros = jnp.zeros((16,), dtype=jnp.float32)

    def zbody(r, _):
        for k in range(D // 16):
            bufs[0][r, pl.ds(k * 16, 16)] = zeros
        return _

    lax.fori_loop(0, ECH, zbody, None)
    off = 0
    for zr in [ECH] * (RPT // ECH) + [RPT % ECH]:
        pltpu.async_copy(bufs[0].at[pl.ds(0, zr)],
                         agg.at[pl.ds(s * RPT + off, zr)], zsem)
        off += zr
    pltpu.sync_copy(src_hbm.at[wid], isrc)
    pltpu.sync_copy(dst_hbm.at[wid], idst)
    for zr in [ECH] * (RPT // ECH) + [RPT % ECH]:
        pltpu.make_async_copy(bufs[0].at[pl.ds(0, zr)],
                              agg.at[pl.ds(s * RPT, zr)], zsem).wait()
    plsc.subcore_barrier()

    def _wait_gather(c_, k):
        pltpu.make_async_copy(hs_hbm.at[isrc.at[c_]], bufs[k], gsems[k]).wait()

    def _start_scatter(c_, k):
        pltpu.async_copy(bufs[k], agg.at[idst.at[c_]], ssems[k], add=True)

    def _wait_scatter_refill(c_, k):
        pltpu.make_async_copy(bufs[k], agg.at[idst.at[c_]], ssems[k]).wait()
        nxt = c_ + NB

        @pl.when(nxt < ECPW)
        def _():
            pltpu.async_copy(hs_hbm.at[isrc.at[nxt]], bufs[k], gsems[k])

    for k in range(NB):
        pltpu.async_copy(hs_hbm.at[isrc.at[k]], bufs[k], gsems[k])

    def body(i, _):
        base = i * NB
        for k in range(NB):
            _wait_gather(base + k, k)
            _start_scatter(base + k, k)
            if k >= NB - 1:
                _wait_scatter_refill(base + k - (NB - 1), k - (NB - 1))
        for k in range(1, NB):
            _wait_scatter_refill(base + k, k)
        return _

    lax.fori_loop(0, EGRP, body, None)

    for k in range(ECPW - EGRP * NB):
        c_ = EGRP * NB + k
        _wait_gather(c_, k)
        pltpu.sync_copy(bufs[k], agg.at[idst.at[c_]], add=True)
    plsc.subcore_barrier()

    pltpu.sync_copy(agg.at[pl.ds(s * RPT, RPT)], out_hbm.at[c, pl.ds(s * RPT, RPT)])


def _mm_scale_body(x_ref, w_ref, deg_ref, out_ref, ns_ref, nd_ref):
    dt = deg_ref[...]
    ds = jnp.maximum(jnp.sum(dt[:, :NW], axis=1, keepdims=True), 1.0)
    dd = jnp.maximum(jnp.sum(dt[:, NW:], axis=1, keepdims=True), 1.0)

    def _rsqrt(d):
        r = jax.lax.rsqrt(d)
        return r * (1.5 - 0.5 * d * r * r)

    ns = _rsqrt(ds)
    nd = _rsqrt(dd)
    ns_ref[...] = ns
    nd_ref[...] = nd
    out_ref[...] = jnp.dot(x_ref[...], w_ref[...], precision=_DEFPREC,
                           preferred_element_type=jnp.float32) * ns


def _mm_scale_tc(x, w, deg_t):
    return pl.pallas_call(
        _mm_scale_body,
        grid=(GRID,),
        in_specs=[
            pl.BlockSpec((BLK, D), lambda i: (i, 0)),
            pl.BlockSpec((D, D), lambda i: (0, 0)),
            pl.BlockSpec((BLK, 2 * NW), lambda i: (i, 0)),
        ],
        out_specs=[
            pl.BlockSpec((BLK, D), lambda i: (i, 0)),
            pl.BlockSpec((BLK, 1), lambda i: (i, 0)),
            pl.BlockSpec((BLK, 1), lambda i: (i, 0)),
        ],
        out_shape=[
            jax.ShapeDtypeStruct((N, D), jnp.float32),
            jax.ShapeDtypeStruct((N, 1), jnp.float32),
            jax.ShapeDtypeStruct((N, 1), jnp.float32),
        ],
    )(x, w, deg_t)


def _mid_body(p_ref, nd_ref, b_ref, w_ref, ns_ref, out_ref):
    agg = p_ref[0] + p_ref[1]
    h = jax.nn.relu(agg * nd_ref[...] + b_ref[...])
    out_ref[...] = jnp.dot(h, w_ref[...], precision=_DEFPREC,
                           preferred_element_type=jnp.float32) * ns_ref[...]


def _mid_tc(p, nd_col, b_row, w, ns_col):
    return pl.pallas_call(
        _mid_body,
        grid=(GRID,),
        in_specs=[
            pl.BlockSpec((NC, BLK, D), lambda i: (0, i, 0)),
            pl.BlockSpec((BLK, 1), lambda i: (i, 0)),
            pl.BlockSpec((1, D), lambda i: (0, 0)),
            pl.BlockSpec((D, D), lambda i: (0, 0)),
            pl.BlockSpec((BLK, 1), lambda i: (i, 0)),
        ],
        out_specs=pl.BlockSpec((BLK, D), lambda i: (i, 0)),
        out_shape=jax.ShapeDtypeStruct((N, D), jnp.float32),
    )(p, nd_col, b_row, w, ns_col)


def _final_body(p_ref, nd_ref, b_ref, wc1_ref, bc1_ref, wc2_ref, bc2_ref,
                ne_ref, ge_ref, lg_ref, acc_ref):
    i = pl.program_id(0)
    agg = p_ref[0] + p_ref[1]
    ne = jax.nn.relu(agg * nd_ref[...] + b_ref[...])
    ne_ref[...] = ne

    @pl.when(i == 0)
    def _():
        acc_ref[...] = jnp.zeros_like(acc_ref)

    acc_ref[...] += jnp.sum(ne, axis=0, keepdims=True)

    @pl.when(i == GRID - 1)
    def _():
        ge = acc_ref[...] * (1.0 / N)
        ge_ref[...] = ge
        hc = jax.nn.relu(jnp.dot(ge, wc1_ref[...], precision=_DEFPREC,
                                 preferred_element_type=jnp.float32)
                         + bc1_ref[...])
        lg_ref[...] = (jnp.sum(hc * wc2_ref[...], axis=1, keepdims=True)
                       + bc2_ref[...])


def _final_tc(p, nd_col, b_row, wc1, bc1_row, wc2, bc2_row):
    return pl.pallas_call(
        _final_body,
        grid=(GRID,),
        in_specs=[
            pl.BlockSpec((NC, BLK, D), lambda i: (0, i, 0)),
            pl.BlockSpec((BLK, 1), lambda i: (i, 0)),
            pl.BlockSpec((1, D), lambda i: (0, 0)),
            pl.BlockSpec((D, D), lambda i: (0, 0)),
            pl.BlockSpec((1, D), lambda i: (0, 0)),
            pl.BlockSpec((1, D), lambda i: (0, 0)),
            pl.BlockSpec((1, 1), lambda i: (0, 0)),
        ],
        out_specs=[
            pl.BlockSpec((BLK, D), lambda i: (i, 0)),
            pl.BlockSpec((1, D), lambda i: (0, 0)),
            pl.BlockSpec((1, 1), lambda i: (0, 0)),
        ],
        out_shape=[
            jax.ShapeDtypeStruct((N, D), jnp.float32),
            jax.ShapeDtypeStruct((1, D), jnp.float32),
            jax.ShapeDtypeStruct((1, 1), jnp.float32),
        ],
        scratch_shapes=[pltpu.VMEM((1, D), jnp.float32)],
    )(p, nd_col, b_row, wc1, bc1_row, wc2, bc2_row)


def kernel(x, edge_index, W1, b1, W2, b2, Wc1, bc1, Wc2, bc2):
    src = edge_index[0]
    dst = edge_index[1]
    src2 = src.reshape(NW, CPW, CH)
    dst2 = dst.reshape(NW, CPW, CH)
    npad_e = EPAD - E
    src3 = jnp.concatenate(
        [src, (jnp.arange(npad_e, dtype=jnp.int32) * 37) % N]
    ).reshape(NW, ECPW, ECH)
    dst3 = jnp.concatenate(
        [dst, N + (jnp.arange(npad_e, dtype=jnp.int32) % (NPAD - N))]
    ).reshape(NW, ECPW, ECH)

    deg = _deg_kernel(src2, dst2)
    deg_t = jnp.transpose(deg, (2, 0, 1)).reshape(N, 2 * NW)
    b1r = b1.reshape(1, D)
    b2r = b2.reshape(1, D)
    bc1r = bc1.reshape(1, D)
    bc2r = bc2.reshape(1, 1)
    wc2r = Wc2.reshape(1, D)

    hs1, ns_col, nd_col = _mm_scale_tc(x, W1, deg_t)
    p1 = _edge_kernel(hs1, src3, dst3)
    hs2 = _mid_tc(p1, nd_col, b1r, W2, ns_col)
    p2 = _edge_kernel(hs2, src3, dst3)
    node_emb, graph_emb, logits = _final_tc(p2, nd_col, b2r, Wc1, bc1r,
                                            wc2r, bc2r)
    return (node_emb, graph_emb, logits)
